# baseline (speedup 1.0000x reference)
import functools

import jax
import jax.numpy as jnp
from jax import lax
from jax.experimental import pallas as pl
from jax.experimental.pallas import tpu as pltpu
from jax.experimental.pallas import tpu_sc as plsc

N_NODES = 10000
N_EDGES = 320000
NFEAT = 128
BN_EPS = 1e-5

NC = 2
NS = 16
NW = NC * NS
EDGES_PER_TILE = N_EDGES // NW
CHUNK = 80
NCHUNK = EDGES_PER_TILE // CHUNK
STRIPE = 640
N_PAD = NS * STRIPE
TAIL = N_NODES - (NS - 1) * STRIPE
ZROWS = 8


def _sc_segment_sum(x, packed_r):
    mesh = plsc.VectorSubcoreMesh(core_axis_name="c", subcore_axis_name="s")

    @functools.partial(
        pl.kernel,
        mesh=mesh,
        out_type=jax.ShapeDtypeStruct((NC, N_NODES, NFEAT), jnp.float32),
        scratch_types=[
            pltpu.VMEM((NCHUNK, CHUNK), jnp.int32),
            pltpu.VMEM((3, CHUNK), jnp.int32),
            pltpu.VMEM((3, CHUNK), jnp.int32),
            pltpu.VMEM((CHUNK, NFEAT), jnp.float32),
            pltpu.VMEM((CHUNK, NFEAT), jnp.float32),
            pltpu.VMEM((CHUNK, NFEAT), jnp.float32),
            pltpu.VMEM((ZROWS, NFEAT), jnp.float32),
            pltpu.VMEM_SHARED((N_PAD, NFEAT), jnp.float32),
            pltpu.SemaphoreType.DMA,
            pltpu.SemaphoreType.DMA,
            pltpu.SemaphoreType.DMA,
            pltpu.SemaphoreType.DMA,
            pltpu.SemaphoreType.DMA,
            pltpu.SemaphoreType.DMA,
        ],
    )
    def seg_sum(x_hbm, pk_hbm, out_hbm, pk_v, src_v, dst_v,
                rows0, rows1, rows2, zbuf, acc,
                gs0, gs1, gs2, ss0, ss1, ss2):
        c = lax.axis_index("c")
        s = lax.axis_index("s")
        wid = c * NS + s
        row0 = s * STRIPE
        last = s == NS - 1

        @pl.when(c == 0)
        def _():
            @pl.when(jnp.logical_not(last))
            def _():
                pltpu.sync_copy(x_hbm.at[pl.ds(row0, STRIPE)],
                                acc.at[pl.ds(row0, STRIPE)])

            @pl.when(last)
            def _():
                pltpu.sync_copy(x_hbm.at[pl.ds(row0, TAIL)],
                                acc.at[pl.ds(row0, TAIL)])

        @pl.when(c != 0)
        def _():
            zv = jnp.zeros((16,), jnp.float32)
            for r in range(ZROWS):
                for j in range(NFEAT // 16):
                    zbuf[r, pl.ds(j * 16, 16)] = zv

            def zb(i, _):
                pltpu.sync_copy(zbuf, acc.at[pl.ds(row0 + i * ZROWS, ZROWS)])
                return 0

            nzb = jnp.where(last, TAIL // ZROWS, STRIPE // ZROWS)
            lax.fori_loop(0, nzb, zb, 0)

        plsc.subcore_barrier()

        pltpu.sync_copy(pk_hbm.at[wid], pk_v)

        rows = (rows0, rows1, rows2)
        gsem = (gs0, gs1, gs2)
        ssem = (ss0, ss1, ss2)

        def unpack(j, b):
            for k in range(CHUNK // 16):
                v = pk_v[j, pl.ds(k * 16, 16)]
                src_v[b, pl.ds(k * 16, 16)] = lax.shift_right_logical(v, 16)
                dst_v[b, pl.ds(k * 16, 16)] = lax.bitwise_and(v, 0xFFFF)

        def start_gather(b):
            pltpu.async_copy(x_hbm.at[src_v.at[b]], rows[b], gsem[b])

        def wait_gather(b):
            pltpu.make_async_copy(x_hbm.at[pl.ds(0, CHUNK)], rows[b], gsem[b]).wait()

        def start_scatter(b):
            pltpu.async_copy(rows[b], acc.at[dst_v.at[b]], ssem[b], add=True)

        def wait_scatter(b):
            pltpu.make_async_copy(rows[b], acc.at[pl.ds(0, CHUNK)], ssem[b]).wait()

        unpack(0, 0)
        start_gather(0)
        unpack(1, 1)
        start_gather(1)

        def triple(i, _):
            for k in range(3):
                j = 3 * i + k
                bp = (k + 2) % 3

                @pl.when(j < NCHUNK)
                def _():
                    wait_gather(k)
                    start_scatter(k)

                @pl.when(j + 2 < NCHUNK)
                def _():
                    @pl.when(j >= 1)
                    def _():
                        wait_scatter(bp)

                    unpack(j + 2, bp)
                    start_gather(bp)

            return 0

        lax.fori_loop(0, (NCHUNK + 2) // 3, triple, 0)

        wait_scatter((NCHUNK - 3) % 3)
        wait_scatter((NCHUNK - 2) % 3)
        wait_scatter((NCHUNK - 1) % 3)

        plsc.subcore_barrier()

        @pl.when(jnp.logical_not(last))
        def _():
            pltpu.sync_copy(acc.at[pl.ds(row0, STRIPE)],
                            out_hbm.at[c, pl.ds(row0, STRIPE)])

        @pl.when(last)
        def _():
            pltpu.sync_copy(acc.at[pl.ds(row0, TAIL)],
                            out_hbm.at[c, pl.ds(row0, TAIL)])

    return seg_sum(x, packed_r)


def _tc_mlp(p, W1, b1, s1, t1, W2, b2, Wfc, bfc):
    R = 2000
    dn = (((1,), (1,)), ((), ()))

    def body(p_ref, w1_ref, b1_ref, s1_ref, t1_ref, w2_ref, b2_ref, wfc_ref,
             bfc_ref, out_ref):
        h = p_ref[0] + p_ref[1]
        z1 = jnp.maximum(
            lax.dot_general(h, w1_ref[...], dn, preferred_element_type=jnp.float32)
            + b1_ref[...], 0.0)
        bn1 = z1 * s1_ref[...] + t1_ref[...]
        z2 = jnp.maximum(
            lax.dot_general(bn1, w2_ref[...], dn, preferred_element_type=jnp.float32)
            + b2_ref[...], 0.0)
        out_ref[...] = jnp.sum(z2 * wfc_ref[...], axis=1, keepdims=True) + bfc_ref[...]

    vec = lambda i: (0, 0)
    return pl.pallas_call(
        body,
        grid=(N_NODES // R,),
        in_specs=[
            pl.BlockSpec((NC, R, NFEAT), lambda i: (0, i, 0)),
            pl.BlockSpec((NFEAT, NFEAT), vec),
            pl.BlockSpec((1, NFEAT), vec),
            pl.BlockSpec((1, NFEAT), vec),
            pl.BlockSpec((1, NFEAT), vec),
            pl.BlockSpec((NFEAT, NFEAT), vec),
            pl.BlockSpec((1, NFEAT), vec),
            pl.BlockSpec((1, NFEAT), vec),
            pl.BlockSpec((1, 1), vec),
        ],
        out_specs=pl.BlockSpec((R, 1), lambda i: (i, 0)),
        out_shape=jax.ShapeDtypeStruct((N_NODES, 1), jnp.float32),
    )(p, W1, b1, s1, t1, W2, b2, Wfc, bfc)


def kernel(x, edge_index, W1, b1, g1, beta1, m1, v1, W2, b2, g2, beta2, m2, v2, Wfc, bfc):
    packed = jnp.bitwise_or(jnp.left_shift(edge_index[0], 16), edge_index[1])
    packed_r = packed.reshape(NW, NCHUNK, CHUNK)
    p = _sc_segment_sum(x, packed_r)

    s1 = (g1 * lax.rsqrt(v1 + BN_EPS)).reshape(1, NFEAT)
    t1 = (beta1 - m1 * g1 * lax.rsqrt(v1 + BN_EPS)).reshape(1, NFEAT)
    s2 = g2 * lax.rsqrt(v2 + BN_EPS)
    t2 = beta2 - m2 * s2
    wfc = Wfc * s2[None, :]
    bfc_f = (Wfc @ t2 + bfc).reshape(1, 1)
    return _tc_mlp(p, W1, b1.reshape(1, NFEAT), s1, t1, W2,
                   b2.reshape(1, NFEAT), wfc, bfc_f)


# speedup vs baseline: 1.0098x; 1.0098x over previous
import functools

import jax
import jax.numpy as jnp
from jax import lax
from jax.experimental import pallas as pl
from jax.experimental.pallas import tpu as pltpu
from jax.experimental.pallas import tpu_sc as plsc

N_NODES = 10000
N_EDGES = 320000
NFEAT = 128
BN_EPS = 1e-5

NC = 2
NS = 16
NW = NC * NS
EDGES_PER_TILE = N_EDGES // NW
CHUNK = 80
NCHUNK = EDGES_PER_TILE // CHUNK
STRIPE = 640
N_PAD = NS * STRIPE
TAIL = N_NODES - (NS - 1) * STRIPE
ZROWS = 8


def _sc_segment_sum(x, packed_r):
    mesh = plsc.VectorSubcoreMesh(core_axis_name="c", subcore_axis_name="s")

    @functools.partial(
        pl.kernel,
        mesh=mesh,
        out_type=jax.ShapeDtypeStruct((NC, N_NODES, NFEAT), jnp.float32),
        scratch_types=[
            pltpu.VMEM((NCHUNK, CHUNK), jnp.int32),
            pltpu.VMEM((3, CHUNK), jnp.int32),
            pltpu.VMEM((3, CHUNK), jnp.int32),
            pltpu.VMEM((CHUNK, NFEAT), jnp.float32),
            pltpu.VMEM((CHUNK, NFEAT), jnp.float32),
            pltpu.VMEM((CHUNK, NFEAT), jnp.float32),
            pltpu.VMEM((ZROWS, NFEAT), jnp.float32),
            pltpu.VMEM_SHARED((N_PAD, NFEAT), jnp.float32),
            pltpu.SemaphoreType.DMA,
            pltpu.SemaphoreType.DMA,
            pltpu.SemaphoreType.DMA,
            pltpu.SemaphoreType.DMA,
            pltpu.SemaphoreType.DMA,
            pltpu.SemaphoreType.DMA,
            pltpu.SemaphoreType.DMA,
        ],
    )
    def seg_sum(x_hbm, pk_hbm, out_hbm, pk_v, src_v, dst_v,
                rows0, rows1, rows2, zbuf, acc,
                gs0, gs1, gs2, ss0, ss1, ss2, isem):
        c = lax.axis_index("c")
        s = lax.axis_index("s")
        wid = c * NS + s
        row0 = s * STRIPE
        last = s == NS - 1

        pltpu.async_copy(pk_hbm.at[wid], pk_v, isem)

        @pl.when(c == 0)
        def _():
            @pl.when(jnp.logical_not(last))
            def _():
                pltpu.sync_copy(x_hbm.at[pl.ds(row0, STRIPE)],
                                acc.at[pl.ds(row0, STRIPE)])

            @pl.when(last)
            def _():
                pltpu.sync_copy(x_hbm.at[pl.ds(row0, TAIL)],
                                acc.at[pl.ds(row0, TAIL)])

        @pl.when(c != 0)
        def _():
            zv = jnp.zeros((16,), jnp.float32)
            for r in range(ZROWS):
                for j in range(NFEAT // 16):
                    zbuf[r, pl.ds(j * 16, 16)] = zv

            def zb(i, _):
                pltpu.sync_copy(zbuf, acc.at[pl.ds(row0 + i * ZROWS, ZROWS)])
                return 0

            nzb = jnp.where(last, TAIL // ZROWS, STRIPE // ZROWS)
            lax.fori_loop(0, nzb, zb, 0)

        plsc.subcore_barrier()

        pltpu.make_async_copy(pk_hbm.at[wid], pk_v, isem).wait()

        rows = (rows0, rows1, rows2)
        gsem = (gs0, gs1, gs2)
        ssem = (ss0, ss1, ss2)

        def unpack(j, b):
            for k in range(CHUNK // 16):
                v = pk_v[j, pl.ds(k * 16, 16)]
                src_v[b, pl.ds(k * 16, 16)] = lax.shift_right_logical(v, 16)
                dst_v[b, pl.ds(k * 16, 16)] = lax.bitwise_and(v, 0xFFFF)

        def start_gather(b):
            pltpu.async_copy(x_hbm.at[src_v.at[b]], rows[b], gsem[b])

        def wait_gather(b):
            pltpu.make_async_copy(x_hbm.at[pl.ds(0, CHUNK)], rows[b], gsem[b]).wait()

        def start_scatter(b):
            pltpu.async_copy(rows[b], acc.at[dst_v.at[b]], ssem[b], add=True)

        def wait_scatter(b):
            pltpu.make_async_copy(rows[b], acc.at[pl.ds(0, CHUNK)], ssem[b]).wait()

        unpack(0, 0)
        start_gather(0)
        unpack(1, 1)
        start_gather(1)

        def triple(i, _):
            for k in range(3):
                j = 3 * i + k
                bp = (k + 2) % 3

                @pl.when(j < NCHUNK)
                def _():
                    wait_gather(k)
                    start_scatter(k)

                @pl.when(j + 2 < NCHUNK)
                def _():
                    @pl.when(j >= 1)
                    def _():
                        wait_scatter(bp)

                    unpack(j + 2, bp)
                    start_gather(bp)

            return 0

        lax.fori_loop(0, (NCHUNK + 2) // 3, triple, 0)

        wait_scatter((NCHUNK - 3) % 3)
        wait_scatter((NCHUNK - 2) % 3)
        wait_scatter((NCHUNK - 1) % 3)

        plsc.subcore_barrier()

        @pl.when(jnp.logical_not(last))
        def _():
            pltpu.sync_copy(acc.at[pl.ds(row0, STRIPE)],
                            out_hbm.at[c, pl.ds(row0, STRIPE)])

        @pl.when(last)
        def _():
            pltpu.sync_copy(acc.at[pl.ds(row0, TAIL)],
                            out_hbm.at[c, pl.ds(row0, TAIL)])

    return seg_sum(x, packed_r)


def _tc_mlp(p, W1, b1, s1, t1, W2, b2, Wfc, bfc):
    R = 2000
    dn = (((1,), (1,)), ((), ()))

    def body(p_ref, w1_ref, b1_ref, s1_ref, t1_ref, w2_ref, b2_ref, wfc_ref,
             bfc_ref, out_ref):
        h = p_ref[0] + p_ref[1]
        z1 = jnp.maximum(
            lax.dot_general(h, w1_ref[...], dn, preferred_element_type=jnp.float32)
            + b1_ref[...], 0.0)
        bn1 = z1 * s1_ref[...] + t1_ref[...]
        z2 = jnp.maximum(
            lax.dot_general(bn1, w2_ref[...], dn, preferred_element_type=jnp.float32)
            + b2_ref[...], 0.0)
        out_ref[...] = jnp.sum(z2 * wfc_ref[...], axis=1, keepdims=True) + bfc_ref[...]

    vec = lambda i: (0, 0)
    return pl.pallas_call(
        body,
        grid=(N_NODES // R,),
        in_specs=[
            pl.BlockSpec((NC, R, NFEAT), lambda i: (0, i, 0)),
            pl.BlockSpec((NFEAT, NFEAT), vec),
            pl.BlockSpec((1, NFEAT), vec),
            pl.BlockSpec((1, NFEAT), vec),
            pl.BlockSpec((1, NFEAT), vec),
            pl.BlockSpec((NFEAT, NFEAT), vec),
            pl.BlockSpec((1, NFEAT), vec),
            pl.BlockSpec((1, NFEAT), vec),
            pl.BlockSpec((1, 1), vec),
        ],
        out_specs=pl.BlockSpec((R, 1), lambda i: (i, 0)),
        out_shape=jax.ShapeDtypeStruct((N_NODES, 1), jnp.float32),
    )(p, W1, b1, s1, t1, W2, b2, Wfc, bfc)


def kernel(x, edge_index, W1, b1, g1, beta1, m1, v1, W2, b2, g2, beta2, m2, v2, Wfc, bfc):
    packed = jnp.bitwise_or(jnp.left_shift(edge_index[0], 16), edge_index[1])
    packed_r = packed.reshape(NW, NCHUNK, CHUNK)
    p = _sc_segment_sum(x, packed_r)

    s1 = (g1 * lax.rsqrt(v1 + BN_EPS)).reshape(1, NFEAT)
    t1 = (beta1 - m1 * g1 * lax.rsqrt(v1 + BN_EPS)).reshape(1, NFEAT)
    s2 = g2 * lax.rsqrt(v2 + BN_EPS)
    t2 = beta2 - m2 * s2
    wfc = Wfc * s2[None, :]
    bfc_f = (Wfc @ t2 + bfc).reshape(1, 1)
    return _tc_mlp(p, W1, b1.reshape(1, NFEAT), s1, t1, W2,
                   b2.reshape(1, NFEAT), wfc, bfc_f)


# speedup vs baseline: 1.0126x; 1.0028x over previous
import functools

import jax
import jax.numpy as jnp
from jax import lax
from jax.experimental import pallas as pl
from jax.experimental.pallas import tpu as pltpu
from jax.experimental.pallas import tpu_sc as plsc

N_NODES = 10000
N_EDGES = 320000
NFEAT = 128
BN_EPS = 1e-5

NC = 2
NS = 16
NW = NC * NS
EDGES_PER_TILE = N_EDGES // NW
CHUNK = 80
NCHUNK = EDGES_PER_TILE // CHUNK
STRIPE = 640
N_PAD = NS * STRIPE
TAIL = N_NODES - (NS - 1) * STRIPE
ZROWS = 8


def _sc_segment_sum(x, packed_r):
    mesh = plsc.VectorSubcoreMesh(core_axis_name="c", subcore_axis_name="s")

    @functools.partial(
        pl.kernel,
        mesh=mesh,
        out_type=jax.ShapeDtypeStruct((NC, N_NODES, NFEAT), jnp.float32),
        scratch_types=[
            pltpu.VMEM((NCHUNK, CHUNK), jnp.int32),
            pltpu.VMEM((3, CHUNK), jnp.int32),
            pltpu.VMEM((3, CHUNK), jnp.int32),
            pltpu.VMEM((CHUNK, NFEAT), jnp.float32),
            pltpu.VMEM((CHUNK, NFEAT), jnp.float32),
            pltpu.VMEM((CHUNK, NFEAT), jnp.float32),
            pltpu.VMEM((ZROWS, NFEAT), jnp.float32),
            pltpu.VMEM_SHARED((N_PAD, NFEAT), jnp.float32),
            pltpu.SemaphoreType.DMA,
            pltpu.SemaphoreType.DMA,
            pltpu.SemaphoreType.DMA,
            pltpu.SemaphoreType.DMA,
            pltpu.SemaphoreType.DMA,
            pltpu.SemaphoreType.DMA,
            pltpu.SemaphoreType.DMA,
            pltpu.SemaphoreType.DMA,
        ],
    )
    def seg_sum(x_hbm, pk_hbm, out_hbm, pk_v, src_v, dst_v,
                rows0, rows1, rows2, zbuf, acc,
                gs0, gs1, gs2, ss0, ss1, ss2, isem, zsem):
        c = lax.axis_index("c")
        s = lax.axis_index("s")
        wid = c * NS + s
        row0 = s * STRIPE
        last = s == NS - 1

        pltpu.async_copy(pk_hbm.at[wid], pk_v, isem)

        @pl.when(c == 0)
        def _():
            @pl.when(jnp.logical_not(last))
            def _():
                pltpu.sync_copy(x_hbm.at[pl.ds(row0, STRIPE)],
                                acc.at[pl.ds(row0, STRIPE)])

            @pl.when(last)
            def _():
                pltpu.sync_copy(x_hbm.at[pl.ds(row0, TAIL)],
                                acc.at[pl.ds(row0, TAIL)])

        @pl.when(c != 0)
        def _():
            zv = jnp.zeros((16,), jnp.float32)
            for r in range(ZROWS):
                for j in range(NFEAT // 16):
                    zbuf[r, pl.ds(j * 16, 16)] = zv

            nzb = jnp.where(last, TAIL // ZROWS, STRIPE // ZROWS)

            def zb(i, _):
                pltpu.async_copy(zbuf, acc.at[pl.ds(row0 + i * ZROWS, ZROWS)], zsem)
                return 0

            lax.fori_loop(0, nzb, zb, 0)

            def zw(i, _):
                pltpu.make_async_copy(zbuf, acc.at[pl.ds(row0, ZROWS)], zsem).wait()
                return 0

            lax.fori_loop(0, nzb, zw, 0)

        plsc.subcore_barrier()

        pltpu.make_async_copy(pk_hbm.at[wid], pk_v, isem).wait()

        rows = (rows0, rows1, rows2)
        gsem = (gs0, gs1, gs2)
        ssem = (ss0, ss1, ss2)

        def unpack(j, b):
            for k in range(CHUNK // 16):
                v = pk_v[j, pl.ds(k * 16, 16)]
                src_v[b, pl.ds(k * 16, 16)] = lax.shift_right_logical(v, 16)
                dst_v[b, pl.ds(k * 16, 16)] = lax.bitwise_and(v, 0xFFFF)

        def start_gather(b):
            pltpu.async_copy(x_hbm.at[src_v.at[b]], rows[b], gsem[b])

        def wait_gather(b):
            pltpu.make_async_copy(x_hbm.at[pl.ds(0, CHUNK)], rows[b], gsem[b]).wait()

        def start_scatter(b):
            pltpu.async_copy(rows[b], acc.at[dst_v.at[b]], ssem[b], add=True)

        def wait_scatter(b):
            pltpu.make_async_copy(rows[b], acc.at[pl.ds(0, CHUNK)], ssem[b]).wait()

        unpack(0, 0)
        start_gather(0)
        unpack(1, 1)
        start_gather(1)

        def triple(i, _):
            for k in range(3):
                j = 3 * i + k
                bp = (k + 2) % 3

                @pl.when(j < NCHUNK)
                def _():
                    wait_gather(k)
                    start_scatter(k)

                @pl.when(j + 2 < NCHUNK)
                def _():
                    @pl.when(j >= 1)
                    def _():
                        wait_scatter(bp)

                    unpack(j + 2, bp)
                    start_gather(bp)

            return 0

        lax.fori_loop(0, (NCHUNK + 2) // 3, triple, 0)

        wait_scatter((NCHUNK - 3) % 3)
        wait_scatter((NCHUNK - 2) % 3)
        wait_scatter((NCHUNK - 1) % 3)

        plsc.subcore_barrier()

        @pl.when(jnp.logical_not(last))
        def _():
            pltpu.sync_copy(acc.at[pl.ds(row0, STRIPE)],
                            out_hbm.at[c, pl.ds(row0, STRIPE)])

        @pl.when(last)
        def _():
            pltpu.sync_copy(acc.at[pl.ds(row0, TAIL)],
                            out_hbm.at[c, pl.ds(row0, TAIL)])

    return seg_sum(x, packed_r)


def _tc_mlp(p, W1, b1, s1, t1, W2, b2, Wfc, bfc):
    R = 2000
    dn = (((1,), (1,)), ((), ()))

    def body(p_ref, w1_ref, b1_ref, s1_ref, t1_ref, w2_ref, b2_ref, wfc_ref,
             bfc_ref, out_ref):
        h = p_ref[0] + p_ref[1]
        z1 = jnp.maximum(
            lax.dot_general(h, w1_ref[...], dn, preferred_element_type=jnp.float32)
            + b1_ref[...], 0.0)
        bn1 = z1 * s1_ref[...] + t1_ref[...]
        z2 = jnp.maximum(
            lax.dot_general(bn1, w2_ref[...], dn, preferred_element_type=jnp.float32)
            + b2_ref[...], 0.0)
        out_ref[...] = jnp.sum(z2 * wfc_ref[...], axis=1, keepdims=True) + bfc_ref[...]

    vec = lambda i: (0, 0)
    return pl.pallas_call(
        body,
        grid=(N_NODES // R,),
        in_specs=[
            pl.BlockSpec((NC, R, NFEAT), lambda i: (0, i, 0)),
            pl.BlockSpec((NFEAT, NFEAT), vec),
            pl.BlockSpec((1, NFEAT), vec),
            pl.BlockSpec((1, NFEAT), vec),
            pl.BlockSpec((1, NFEAT), vec),
            pl.BlockSpec((NFEAT, NFEAT), vec),
            pl.BlockSpec((1, NFEAT), vec),
            pl.BlockSpec((1, NFEAT), vec),
            pl.BlockSpec((1, 1), vec),
        ],
        out_specs=pl.BlockSpec((R, 1), lambda i: (i, 0)),
        out_shape=jax.ShapeDtypeStruct((N_NODES, 1), jnp.float32),
    )(p, W1, b1, s1, t1, W2, b2, Wfc, bfc)


def kernel(x, edge_index, W1, b1, g1, beta1, m1, v1, W2, b2, g2, beta2, m2, v2, Wfc, bfc):
    packed = jnp.bitwise_or(jnp.left_shift(edge_index[0], 16), edge_index[1])
    packed_r = packed.reshape(NW, NCHUNK, CHUNK)
    p = _sc_segment_sum(x, packed_r)

    s1 = (g1 * lax.rsqrt(v1 + BN_EPS)).reshape(1, NFEAT)
    t1 = (beta1 - m1 * g1 * lax.rsqrt(v1 + BN_EPS)).reshape(1, NFEAT)
    s2 = g2 * lax.rsqrt(v2 + BN_EPS)
    t2 = beta2 - m2 * s2
    wfc = Wfc * s2[None, :]
    bfc_f = (Wfc @ t2 + bfc).reshape(1, 1)
    return _tc_mlp(p, W1, b1.reshape(1, NFEAT), s1, t1, W2,
                   b2.reshape(1, NFEAT), wfc, bfc_f)


# speedup vs baseline: 1.0128x; 1.0002x over previous
import functools

import jax
import jax.numpy as jnp
from jax import lax
from jax.experimental import pallas as pl
from jax.experimental.pallas import tpu as pltpu
from jax.experimental.pallas import tpu_sc as plsc

N_NODES = 10000
N_EDGES = 320000
NFEAT = 128
BN_EPS = 1e-5

NC = 2
NS = 16
NW = NC * NS
EDGES_PER_TILE = N_EDGES // NW
CHUNK = 80
NCHUNK = EDGES_PER_TILE // CHUNK
STRIPE = 640
N_PAD = NS * STRIPE
TAIL = N_NODES - (NS - 1) * STRIPE
ZROWS = 8


def _sc_segment_sum(x, packed_r):
    mesh = plsc.VectorSubcoreMesh(core_axis_name="c", subcore_axis_name="s")

    @functools.partial(
        pl.kernel,
        mesh=mesh,
        out_type=jax.ShapeDtypeStruct((N_NODES, NC * NFEAT), jnp.float32),
        scratch_types=[
            pltpu.VMEM((NCHUNK, CHUNK), jnp.int32),
            pltpu.VMEM((3, CHUNK), jnp.int32),
            pltpu.VMEM((3, CHUNK), jnp.int32),
            pltpu.VMEM((CHUNK, NFEAT), jnp.float32),
            pltpu.VMEM((CHUNK, NFEAT), jnp.float32),
            pltpu.VMEM((CHUNK, NFEAT), jnp.float32),
            pltpu.VMEM((ZROWS, NFEAT), jnp.float32),
            pltpu.VMEM_SHARED((N_PAD, NFEAT), jnp.float32),
            pltpu.SemaphoreType.DMA,
            pltpu.SemaphoreType.DMA,
            pltpu.SemaphoreType.DMA,
            pltpu.SemaphoreType.DMA,
            pltpu.SemaphoreType.DMA,
            pltpu.SemaphoreType.DMA,
            pltpu.SemaphoreType.DMA,
            pltpu.SemaphoreType.DMA,
        ],
    )
    def seg_sum(x_hbm, pk_hbm, out_hbm, pk_v, src_v, dst_v,
                rows0, rows1, rows2, zbuf, acc,
                gs0, gs1, gs2, ss0, ss1, ss2, isem, zsem):
        c = lax.axis_index("c")
        s = lax.axis_index("s")
        wid = c * NS + s
        row0 = s * STRIPE
        last = s == NS - 1

        pltpu.async_copy(pk_hbm.at[wid], pk_v, isem)

        @pl.when(c == 0)
        def _():
            @pl.when(jnp.logical_not(last))
            def _():
                pltpu.sync_copy(x_hbm.at[pl.ds(row0, STRIPE)],
                                acc.at[pl.ds(row0, STRIPE)])

            @pl.when(last)
            def _():
                pltpu.sync_copy(x_hbm.at[pl.ds(row0, TAIL)],
                                acc.at[pl.ds(row0, TAIL)])

        @pl.when(c != 0)
        def _():
            zv = jnp.zeros((16,), jnp.float32)
            for r in range(ZROWS):
                for j in range(NFEAT // 16):
                    zbuf[r, pl.ds(j * 16, 16)] = zv

            nzb = jnp.where(last, TAIL // ZROWS, STRIPE // ZROWS)

            def zb(i, _):
                pltpu.async_copy(zbuf, acc.at[pl.ds(row0 + i * ZROWS, ZROWS)], zsem)
                return 0

            lax.fori_loop(0, nzb, zb, 0)

            def zw(i, _):
                pltpu.make_async_copy(zbuf, acc.at[pl.ds(row0, ZROWS)], zsem).wait()
                return 0

            lax.fori_loop(0, nzb, zw, 0)

        plsc.subcore_barrier()

        pltpu.make_async_copy(pk_hbm.at[wid], pk_v, isem).wait()

        rows = (rows0, rows1, rows2)
        gsem = (gs0, gs1, gs2)
        ssem = (ss0, ss1, ss2)

        def unpack(j, b):
            for k in range(CHUNK // 16):
                v = pk_v[j, pl.ds(k * 16, 16)]
                src_v[b, pl.ds(k * 16, 16)] = lax.shift_right_logical(v, 16)
                dst_v[b, pl.ds(k * 16, 16)] = lax.bitwise_and(v, 0xFFFF)

        def start_gather(b):
            pltpu.async_copy(x_hbm.at[src_v.at[b]], rows[b], gsem[b])

        def wait_gather(b):
            pltpu.make_async_copy(x_hbm.at[pl.ds(0, CHUNK)], rows[b], gsem[b]).wait()

        def start_scatter(b):
            pltpu.async_copy(rows[b], acc.at[dst_v.at[b]], ssem[b], add=True)

        def wait_scatter(b):
            pltpu.make_async_copy(rows[b], acc.at[pl.ds(0, CHUNK)], ssem[b]).wait()

        unpack(0, 0)
        start_gather(0)
        unpack(1, 1)
        start_gather(1)

        def triple(i, _):
            for k in range(3):
                j = 3 * i + k
                bp = (k + 2) % 3

                @pl.when(j < NCHUNK)
                def _():
                    wait_gather(k)
                    start_scatter(k)

                @pl.when(j + 2 < NCHUNK)
                def _():
                    @pl.when(j >= 1)
                    def _():
                        wait_scatter(bp)

                    unpack(j + 2, bp)
                    start_gather(bp)

            return 0

        lax.fori_loop(0, (NCHUNK + 2) // 3, triple, 0)

        wait_scatter((NCHUNK - 3) % 3)
        wait_scatter((NCHUNK - 2) % 3)
        wait_scatter((NCHUNK - 1) % 3)

        plsc.subcore_barrier()

        def wb(col0):
            @pl.when(jnp.logical_not(last))
            def _():
                pltpu.sync_copy(acc.at[pl.ds(row0, STRIPE)],
                                out_hbm.at[pl.ds(row0, STRIPE), pl.ds(col0, NFEAT)])

            @pl.when(last)
            def _():
                pltpu.sync_copy(acc.at[pl.ds(row0, TAIL)],
                                out_hbm.at[pl.ds(row0, TAIL), pl.ds(col0, NFEAT)])

        @pl.when(c == 0)
        def _():
            wb(0)

        @pl.when(c != 0)
        def _():
            wb(NFEAT)

    return seg_sum(x, packed_r)


def _tc_mlp(p, W1, b1, s1, t1, W2, b2, Wfc, bfc):
    R = 2000
    dn = (((1,), (1,)), ((), ()))

    def body(p_ref, w1_ref, b1_ref, s1_ref, t1_ref, w2_ref, b2_ref, wfc_ref,
             bfc_ref, out_ref):
        h = p_ref[:, :NFEAT] + p_ref[:, NFEAT:]
        z1 = jnp.maximum(
            lax.dot_general(h, w1_ref[...], dn, preferred_element_type=jnp.float32)
            + b1_ref[...], 0.0)
        bn1 = z1 * s1_ref[...] + t1_ref[...]
        z2 = jnp.maximum(
            lax.dot_general(bn1, w2_ref[...], dn, preferred_element_type=jnp.float32)
            + b2_ref[...], 0.0)
        out_ref[...] = jnp.sum(z2 * wfc_ref[...], axis=1, keepdims=True) + bfc_ref[...]

    vec = lambda i: (0, 0)
    return pl.pallas_call(
        body,
        grid=(N_NODES // R,),
        in_specs=[
            pl.BlockSpec((R, NC * NFEAT), lambda i: (i, 0)),
            pl.BlockSpec((NFEAT, NFEAT), vec),
            pl.BlockSpec((1, NFEAT), vec),
            pl.BlockSpec((1, NFEAT), vec),
            pl.BlockSpec((1, NFEAT), vec),
            pl.BlockSpec((NFEAT, NFEAT), vec),
            pl.BlockSpec((1, NFEAT), vec),
            pl.BlockSpec((1, NFEAT), vec),
            pl.BlockSpec((1, 1), vec),
        ],
        out_specs=pl.BlockSpec((R, 1), lambda i: (i, 0)),
        out_shape=jax.ShapeDtypeStruct((N_NODES, 1), jnp.float32),
    )(p, W1, b1, s1, t1, W2, b2, Wfc, bfc)


def kernel(x, edge_index, W1, b1, g1, beta1, m1, v1, W2, b2, g2, beta2, m2, v2, Wfc, bfc):
    packed = jnp.bitwise_or(jnp.left_shift(edge_index[0], 16), edge_index[1])
    packed_r = packed.reshape(NW, NCHUNK, CHUNK)
    p = _sc_segment_sum(x, packed_r)

    s1 = (g1 * lax.rsqrt(v1 + BN_EPS)).reshape(1, NFEAT)
    t1 = (beta1 - m1 * g1 * lax.rsqrt(v1 + BN_EPS)).reshape(1, NFEAT)
    s2 = g2 * lax.rsqrt(v2 + BN_EPS)
    t2 = beta2 - m2 * s2
    wfc = Wfc * s2[None, :]
    bfc_f = (Wfc @ t2 + bfc).reshape(1, 1)
    return _tc_mlp(p, W1, b1.reshape(1, NFEAT), s1, t1, W2,
                   b2.reshape(1, NFEAT), wfc, bfc_f)
